# Initial kernel scaffold; baseline (speedup 1.0000x reference)
import jax
import jax.numpy as jnp
from jax.experimental import pallas as pl


def kernel(pix_to_face, bary_coords, verts, faces, cam_origin):
    raise NotImplementedError("write your pallas kernel here")


# speedup vs baseline: 12.3943x; 12.3943x over previous
import functools

import jax
import jax.numpy as jnp
from jax import lax
from jax.experimental import pallas as pl
from jax.experimental.pallas import tpu as pltpu
from jax.experimental.pallas import tpu_sc as plsc

_SC_PARAMS = pltpu.CompilerParams(
    use_tc_tiling_on_sc=False, needs_layout_passes=False)

NC = 2
NS = 16
NW = NC * NS
L = 16

_MAGIC = 0x5F3759DF


def _c16(v, dtype=jnp.int32):
    return jnp.full((L,), v, dtype)


def _rsqrt_clamped(ss):
    x = jnp.maximum(ss, _c16(1e-24, jnp.float32))
    i = plsc.bitcast(x, jnp.int32)
    i = _c16(_MAGIC) - lax.shift_right_logical(i, 1)
    y = plsc.bitcast(i, jnp.float32)
    xh = x * _c16(0.5, jnp.float32)
    th = _c16(1.5, jnp.float32)
    y = y * (th - xh * y * y)
    y = y * (th - xh * y * y)
    y = y * (th - xh * y * y)
    return jnp.minimum(y, _c16(1e12, jnp.float32))


def _build_table(V, F_pad):
    FT = F_pad // NW
    SUB = 896
    NSUB = FT // SUB
    NIDX = SUB * 3
    mesh = plsc.VectorSubcoreMesh(
        core_axis_name="c", subcore_axis_name="s", num_cores=NC, num_subcores=NS)

    @functools.partial(
        pl.kernel,
        out_type=jax.ShapeDtypeStruct((F_pad, 16), jnp.float32),
        mesh=mesh,
        compiler_params=_SC_PARAMS,
        scratch_types=[
            pltpu.VMEM((NIDX,), jnp.int32),
            pltpu.VMEM((NIDX, 8), jnp.float32),
            pltpu.VMEM((SUB, 16), jnp.float32),
            pltpu.SemaphoreType.DMA,
        ],
    )
    def build(verts_hbm, fidx_hbm, table_hbm, fidx_v, gath_v, row_v, sem):
        wid = lax.axis_index("s") * NC + lax.axis_index("c")
        iota = lax.iota(jnp.int32, L)
        iota3 = iota * 3
        cols = [_c16(m) for m in range(3)]
        slots = [_c16(si) for si in range(12)]

        @pl.loop(0, NSUB)
        def _sub(s):
            f0 = wid * FT + s * SUB
            pltpu.sync_copy(fidx_hbm.at[pl.ds(f0 * 3, NIDX)], fidx_v)
            descs = [
                pltpu.async_copy(verts_hbm.at[fidx_v.at[pl.ds(j * 128, 128)]],
                                 gath_v.at[pl.ds(j * 128, 128)], sem)
                for j in range(NIDX // 128)
            ]
            for d in descs:
                d.wait()

            @pl.loop(0, SUB // L)
            def _g(g):
                base = g * L
                r3 = iota3 + base * 3
                v = [[plsc.load_gather(gath_v, [r3 + j, cols[m]])
                      for m in range(3)] for j in range(3)]
                e1 = [v[1][m] - v[0][m] for m in range(3)]
                e2 = [v[2][m] - v[0][m] for m in range(3)]
                nx = e1[1] * e2[2] - e1[2] * e2[1]
                ny = e1[2] * e2[0] - e1[0] * e2[2]
                nz = e1[0] * e2[1] - e1[1] * e2[0]
                r = _rsqrt_clamped(nx * nx + ny * ny + nz * nz)
                vals = (v[0][0], v[0][1], v[0][2],
                        v[1][0], v[1][1], v[1][2],
                        v[2][0], v[2][1], v[2][2],
                        nx * r, ny * r, nz * r)
                rows = iota + base
                for si in range(12):
                    plsc.store_scatter(row_v, [rows, slots[si]], vals[si])

            pltpu.sync_copy(row_v, table_hbm.at[pl.ds(f0, SUB)])

    return build


def _shade(F_pad, NB, NPIX):
    PT = NPIX // NW
    CP = 1024
    CH = CP * 3
    NCHUNK = PT // CP
    mesh = plsc.VectorSubcoreMesh(
        core_axis_name="c", subcore_axis_name="s", num_cores=NC, num_subcores=NS)
    out_sds = jax.ShapeDtypeStruct((NPIX,), jnp.float32)

    @functools.partial(
        pl.kernel,
        out_type=(out_sds, out_sds, out_sds),
        mesh=mesh,
        compiler_params=_SC_PARAMS,
        scratch_types=[
            pltpu.VMEM((CH,), jnp.int32),
            pltpu.VMEM((CH, 16), jnp.float32),
            pltpu.VMEM((CH, 3), jnp.float32),
            pltpu.VMEM((3, CP), jnp.float32),
            pltpu.VMEM((L,), jnp.float32),
            pltpu.SemaphoreType.DMA,
        ],
    )
    def shade(table_hbm, pix_hbm, bary_hbm, cam_hbm,
              o0, o1, o2, pix_v, rows_v, bary_v, out_v, cam_v, sem):
        wid = lax.axis_index("s") * NC + lax.axis_index("c")
        iota = lax.iota(jnp.int32, L)
        iota3 = iota * 3
        cols = [_c16(m) for m in range(3)]
        slots = [_c16(si) for si in range(12)]
        outs = (o0, o1, o2)

        pltpu.sync_copy(cam_hbm, cam_v)
        b = wid // (NW // NB)
        cam = [plsc.load_gather(
                   cam_v, [jnp.broadcast_to(b * 3 + c, (L,)).astype(jnp.int32)])
               for c in range(3)]

        @pl.loop(0, NCHUNK)
        def _chunk(s):
            p0 = wid * PT + s * CP
            h0 = p0 * 3
            pltpu.sync_copy(pix_hbm.at[pl.ds(h0, CH)], pix_v)
            descs = [
                pltpu.async_copy(table_hbm.at[pix_v.at[pl.ds(j * 128, 128)]],
                                 rows_v.at[pl.ds(j * 128, 128)], sem)
                for j in range(CH // 128)
            ]
            pltpu.sync_copy(bary_hbm.at[pl.ds(h0, CH)], bary_v)
            for d in descs:
                d.wait()

            @pl.loop(0, CP // L)
            def _g(g):
                for k in range(3):
                    hrow = iota3 + (g * 48 + k)
                    c12 = [plsc.load_gather(rows_v, [hrow, slots[si]])
                           for si in range(12)]
                    bw = [plsc.load_gather(bary_v, [hrow, cols[c]])
                          for c in range(3)]
                    d = []
                    for m in range(3):
                        pm = (bw[0] * c12[m] + bw[1] * c12[3 + m]
                              + bw[2] * c12[6 + m])
                        d.append(pm - cam[m])
                    num = c12[9] * d[0] + c12[10] * d[1] + c12[11] * d[2]
                    ss = d[0] * d[0] + d[1] * d[1] + d[2] * d[2]
                    out_v[k, pl.ds(g * L, L)] = num * _rsqrt_clamped(ss)

            for k in range(3):
                pltpu.sync_copy(out_v.at[k], outs[k].at[pl.ds(p0, CP)])

    return shade


def kernel(pix_to_face, bary_coords, verts, faces, cam_origin):
    N, H, W, K = pix_to_face.shape
    V = verts.shape[0]
    F = faces.shape[0]
    NPIX = N * H * W
    HITS = NPIX * K

    gran = NW * 128
    F_pad = ((F + gran - 1) // gran) * gran
    fidx = jnp.zeros((F_pad * 3,), jnp.int32).at[: F * 3].set(faces.reshape(-1))
    pix = pix_to_face.reshape(HITS)
    bary = bary_coords.reshape(HITS, 3)
    cam = jnp.zeros((L,), jnp.float32).at[: N * 3].set(cam_origin.reshape(-1))

    verts_pad = jnp.pad(verts, ((0, 0), (0, 5)))
    table = _build_table(V, F_pad)(verts_pad, fidx)
    o0, o1, o2 = _shade(F_pad, N, NPIX)(table, pix, bary, cam)
    return tuple(o.reshape(N, H, W, 1) for o in (o0, o1, o2))


# speedup vs baseline: 118.8824x; 9.5917x over previous
import functools

import jax
import jax.numpy as jnp
from jax import lax
from jax.experimental import pallas as pl
from jax.experimental.pallas import tpu as pltpu
from jax.experimental.pallas import tpu_sc as plsc

_SC_PARAMS = pltpu.CompilerParams(
    use_tc_tiling_on_sc=False, needs_layout_passes=False)

NC = 2
NS = 16
NW = NC * NS
L = 16

_MAGIC = 0x5F3759DF


def _c16(v, dtype=jnp.int32):
    return jnp.full((L,), v, dtype)


def _rsqrt_clamped(ss):
    x = jnp.maximum(ss, _c16(1e-24, jnp.float32))
    i = plsc.bitcast(x, jnp.int32)
    i = _c16(_MAGIC) - lax.shift_right_logical(i, 1)
    y = plsc.bitcast(i, jnp.float32)
    xh = x * _c16(0.5, jnp.float32)
    th = _c16(1.5, jnp.float32)
    y = y * (th - xh * y * y)
    y = y * (th - xh * y * y)
    y = y * (th - xh * y * y)
    return jnp.minimum(y, _c16(1e12, jnp.float32))


def _build_table(V, F_pad):
    FT = F_pad // NW
    SUB = 896
    NSUB = FT // SUB
    NIDX = SUB * 3
    mesh = plsc.VectorSubcoreMesh(
        core_axis_name="c", subcore_axis_name="s", num_cores=NC, num_subcores=NS)

    @functools.partial(
        pl.kernel,
        out_type=jax.ShapeDtypeStruct((F_pad, 16), jnp.float32),
        mesh=mesh,
        compiler_params=_SC_PARAMS,
        scratch_types=[
            pltpu.VMEM((NIDX,), jnp.int32),
            pltpu.VMEM((NIDX, 8), jnp.float32),
            pltpu.VMEM((SUB, 16), jnp.float32),
            pltpu.SemaphoreType.DMA,
        ],
    )
    def build(verts_hbm, fidx_hbm, table_hbm, fidx_v, gath_v, row_v, sem):
        wid = lax.axis_index("s") * NC + lax.axis_index("c")
        iota = lax.iota(jnp.int32, L)
        iota3 = iota * 3
        cols = [_c16(m) for m in range(3)]
        slots = [_c16(si) for si in range(12)]

        @pl.loop(0, NSUB)
        def _sub(s):
            f0 = wid * FT + s * SUB
            pltpu.sync_copy(fidx_hbm.at[pl.ds(f0 * 3, NIDX)], fidx_v)
            descs = [
                pltpu.async_copy(verts_hbm.at[fidx_v.at[pl.ds(j * 128, 128)]],
                                 gath_v.at[pl.ds(j * 128, 128)], sem)
                for j in range(NIDX // 128)
            ]
            for d in descs:
                d.wait()

            @pl.loop(0, SUB // L)
            def _g(g):
                base = g * L
                r3 = iota3 + base * 3
                v = [[plsc.load_gather(gath_v, [r3 + j, cols[m]])
                      for m in range(3)] for j in range(3)]
                e1 = [v[1][m] - v[0][m] for m in range(3)]
                e2 = [v[2][m] - v[0][m] for m in range(3)]
                nx = e1[1] * e2[2] - e1[2] * e2[1]
                ny = e1[2] * e2[0] - e1[0] * e2[2]
                nz = e1[0] * e2[1] - e1[1] * e2[0]
                r = _rsqrt_clamped(nx * nx + ny * ny + nz * nz)
                vals = (v[0][0], v[0][1], v[0][2],
                        v[1][0], v[1][1], v[1][2],
                        v[2][0], v[2][1], v[2][2],
                        nx * r, ny * r, nz * r)
                rows = iota + base
                for si in range(12):
                    plsc.store_scatter(row_v, [rows, slots[si]], vals[si])

            pltpu.sync_copy(row_v, table_hbm.at[pl.ds(f0, SUB)])

    return build


def _shade(F_pad, N, K, H, W):
    NPIX = N * H * W
    RB = H // 8
    WC = W // 128
    CP = 8 * W
    UNITS = N * RB * K
    UPT = UNITS // NW
    mesh = plsc.VectorSubcoreMesh(
        core_axis_name="c", subcore_axis_name="s", num_cores=NC, num_subcores=NS)
    out_sds = jax.ShapeDtypeStruct((NPIX,), jnp.float32)

    @functools.partial(
        pl.kernel,
        out_type=(out_sds, out_sds, out_sds),
        mesh=mesh,
        compiler_params=_SC_PARAMS,
        scratch_types=[
            pltpu.VMEM((WC, 8, 128), jnp.int32),
            pltpu.VMEM((CP, 16), jnp.float32),
            pltpu.VMEM((3, WC, 8, 128), jnp.float32),
            pltpu.VMEM((CP,), jnp.float32),
            pltpu.VMEM((L,), jnp.float32),
            pltpu.SemaphoreType.DMA,
        ],
    )
    def shade(table_hbm, pix_hbm, bary_hbm, cam_hbm,
              o0, o1, o2, pix_v, rows_v, bary_v, out_v, cam_v, sem):
        wid = lax.axis_index("s") * NC + lax.axis_index("c")
        iota = lax.iota(jnp.int32, L)
        slots = [_c16(si) for si in range(12)]
        outs = (o0, o1, o2)
        pltpu.sync_copy(cam_hbm, cam_v)

        @pl.loop(0, UPT)
        def _unit(s):
            u = wid * UPT + s
            n = u // (RB * K)
            rem = u - n * (RB * K)
            r = rem // K
            k = rem - r * K
            cam = [plsc.load_gather(
                       cam_v,
                       [jnp.broadcast_to(n * 3 + c, (L,)).astype(jnp.int32)])
                   for c in range(3)]

            pltpu.sync_copy(pix_hbm.at[n, k, r], pix_v)
            for c in range(3):
                pltpu.sync_copy(bary_hbm.at[n, k, c, r], bary_v.at[c])
            descs = []
            for ci in range(WC):
                for ri in range(8):
                    descs.append(pltpu.async_copy(
                        table_hbm.at[pix_v.at[ci, ri]],
                        rows_v.at[pl.ds((ci * 8 + ri) * 128, 128)], sem))
            for d in descs:
                d.wait()

            @pl.loop(0, WC * 8)
            def _g(i):
                ci = i // 8
                ri = i - ci * 8
                outb = ri * W + ci * 128
                for cb in range(8):
                    rows = iota + (i * 128 + cb * 16)
                    c12 = [plsc.load_gather(rows_v, [rows, slots[si]])
                           for si in range(12)]
                    bw = [bary_v[c, ci, ri, pl.ds(cb * 16, L)]
                          for c in range(3)]
                    d = []
                    for m in range(3):
                        pm = (bw[0] * c12[m] + bw[1] * c12[3 + m]
                              + bw[2] * c12[6 + m])
                        d.append(pm - cam[m])
                    num = c12[9] * d[0] + c12[10] * d[1] + c12[11] * d[2]
                    ss = d[0] * d[0] + d[1] * d[1] + d[2] * d[2]
                    out_v[pl.ds(outb + cb * 16, L)] = num * _rsqrt_clamped(ss)

            p0 = n * (H * W) + r * CP
            for kk in range(3):
                @pl.when(k == kk)
                def _store():
                    pltpu.sync_copy(out_v, outs[kk].at[pl.ds(p0, CP)])

    return shade


def kernel(pix_to_face, bary_coords, verts, faces, cam_origin):
    N, H, W, K = pix_to_face.shape
    V = verts.shape[0]
    F = faces.shape[0]
    NPIX = N * H * W

    gran = NW * 128
    F_pad = ((F + gran - 1) // gran) * gran
    fidx = jnp.zeros((F_pad * 3,), jnp.int32).at[: F * 3].set(faces.reshape(-1))
    cam = jnp.zeros((L,), jnp.float32).at[: N * 3].set(cam_origin.reshape(-1))

    pix6 = (pix_to_face.transpose(0, 3, 1, 2)
            .reshape(N, K, H // 8, 8, W // 128, 128)
            .transpose(0, 1, 2, 4, 3, 5))
    bary7 = (bary_coords.transpose(0, 3, 4, 1, 2)
             .reshape(N, K, 3, H // 8, 8, W // 128, 128)
             .transpose(0, 1, 2, 3, 5, 4, 6))

    verts_pad = jnp.pad(verts, ((0, 0), (0, 5)))
    table = _build_table(V, F_pad)(verts_pad, fidx)
    o0, o1, o2 = _shade(F_pad, N, K, H, W)(table, pix6, bary7, cam)
    return tuple(o.reshape(N, H, W, 1) for o in (o0, o1, o2))


# speedup vs baseline: 183.7209x; 1.5454x over previous
import functools

import jax
import jax.numpy as jnp
from jax import lax
from jax.experimental import pallas as pl
from jax.experimental.pallas import tpu as pltpu
from jax.experimental.pallas import tpu_sc as plsc

_SC_PARAMS = pltpu.CompilerParams(
    use_tc_tiling_on_sc=False, needs_layout_passes=False)

NC = 2
NS = 16
NW = NC * NS
L = 16

_MAGIC = 0x5F3759DF


def _c16(v, dtype=jnp.int32):
    return jnp.full((L,), v, dtype)


def _rsqrt_clamped(ss):
    x = jnp.maximum(ss, _c16(1e-24, jnp.float32))
    i = plsc.bitcast(x, jnp.int32)
    i = _c16(_MAGIC) - lax.shift_right_logical(i, 1)
    y = plsc.bitcast(i, jnp.float32)
    xh = x * _c16(0.5, jnp.float32)
    th = _c16(1.5, jnp.float32)
    y = y * (th - xh * y * y)
    y = y * (th - xh * y * y)
    y = y * (th - xh * y * y)
    return jnp.minimum(y, _c16(1e12, jnp.float32))


def _vert8(VB):
    BT = VB // NW
    mesh = plsc.VectorSubcoreMesh(
        core_axis_name="c", subcore_axis_name="s", num_cores=NC, num_subcores=NS)

    @functools.partial(
        pl.kernel,
        out_type=jax.ShapeDtypeStruct((VB * 128, 8), jnp.float32),
        mesh=mesh,
        compiler_params=_SC_PARAMS,
        scratch_types=[
            pltpu.VMEM((BT, 4, 128), jnp.float32),
            pltpu.VMEM((BT * 128, 8), jnp.float32),
        ],
    )
    def relayout(vblk_hbm, v8_hbm, blk_v, out_v):
        wid = lax.axis_index("s") * NC + lax.axis_index("c")
        iota = lax.iota(jnp.int32, L)
        cols = [_c16(m) for m in range(3)]
        b0 = wid * BT
        pltpu.sync_copy(vblk_hbm.at[pl.ds(b0, BT)], blk_v)

        @pl.loop(0, BT * 8)
        def _g(g):
            blk = g // 8
            cb = g - blk * 8
            rows = iota + g * L
            for j in range(3):
                val = blk_v[blk, j, pl.ds(cb * L, L)]
                plsc.store_scatter(out_v, [rows, cols[j]], val)

        pltpu.sync_copy(out_v, v8_hbm.at[pl.ds(b0 * 128, BT * 128)])

    return relayout


def _build_table(V, F_pad):
    FT = F_pad // NW
    SUB = 896
    NSUB = FT // SUB
    SUBB = SUB // 128
    NIDX = SUB * 3
    mesh = plsc.VectorSubcoreMesh(
        core_axis_name="c", subcore_axis_name="s", num_cores=NC, num_subcores=NS)

    @functools.partial(
        pl.kernel,
        out_type=jax.ShapeDtypeStruct((F_pad, 16), jnp.float32),
        mesh=mesh,
        compiler_params=_SC_PARAMS,
        scratch_types=[
            pltpu.VMEM((SUBB, 4, 128), jnp.int32),
            pltpu.VMEM((NIDX, 8), jnp.float32),
            pltpu.VMEM((SUB, 16), jnp.float32),
            pltpu.SemaphoreType.DMA,
        ],
    )
    def build(verts_hbm, fidx_hbm, table_hbm, fidx_v, gath_v, row_v, sem):
        wid = lax.axis_index("s") * NC + lax.axis_index("c")
        iota = lax.iota(jnp.int32, L)
        cols = [_c16(m) for m in range(3)]
        slots = [_c16(si) for si in range(12)]

        @pl.loop(0, NSUB)
        def _sub(s):
            f0 = wid * FT + s * SUB
            b0 = f0 // 128
            pltpu.sync_copy(fidx_hbm.at[pl.ds(b0, SUBB)], fidx_v)
            descs = [
                pltpu.async_copy(verts_hbm.at[fidx_v.at[fb, j]],
                                 gath_v.at[pl.ds((fb * 3 + j) * 128, 128)],
                                 sem)
                for fb in range(SUBB) for j in range(3)
            ]
            for d in descs:
                d.wait()

            @pl.loop(0, SUB // L)
            def _g(g):
                base = g * L
                fb = g // 8
                cb = g - fb * 8
                r0 = fb * 384 + cb * L
                v = [[plsc.load_gather(gath_v, [iota + (r0 + j * 128), cols[m]])
                      for m in range(3)] for j in range(3)]
                e1 = [v[1][m] - v[0][m] for m in range(3)]
                e2 = [v[2][m] - v[0][m] for m in range(3)]
                nx = e1[1] * e2[2] - e1[2] * e2[1]
                ny = e1[2] * e2[0] - e1[0] * e2[2]
                nz = e1[0] * e2[1] - e1[1] * e2[0]
                r = _rsqrt_clamped(nx * nx + ny * ny + nz * nz)
                vals = (v[0][0], v[0][1], v[0][2],
                        v[1][0], v[1][1], v[1][2],
                        v[2][0], v[2][1], v[2][2],
                        nx * r, ny * r, nz * r)
                rows = iota + base
                for si in range(12):
                    plsc.store_scatter(row_v, [rows, slots[si]], vals[si])

            pltpu.sync_copy(row_v, table_hbm.at[pl.ds(f0, SUB)])

    return build


def _shade(F_pad, N, K, H, W):
    NPIX = N * H * W
    RB = H // 8
    WC = W // 128
    CP = 8 * W
    UNITS = N * RB * K
    UPT = UNITS // NW
    mesh = plsc.VectorSubcoreMesh(
        core_axis_name="c", subcore_axis_name="s", num_cores=NC, num_subcores=NS)
    out_sds = jax.ShapeDtypeStruct((NPIX,), jnp.float32)

    @functools.partial(
        pl.kernel,
        out_type=(out_sds, out_sds, out_sds),
        mesh=mesh,
        compiler_params=_SC_PARAMS,
        scratch_types=[
            pltpu.VMEM((WC, 8, 128), jnp.int32),
            pltpu.VMEM((CP, 16), jnp.float32),
            pltpu.VMEM((3, WC, 8, 128), jnp.float32),
            pltpu.VMEM((CP,), jnp.float32),
            pltpu.VMEM((L,), jnp.float32),
            pltpu.SemaphoreType.DMA,
        ],
    )
    def shade(table_hbm, pix_hbm, bary_hbm, cam_hbm,
              o0, o1, o2, pix_v, rows_v, bary_v, out_v, cam_v, sem):
        wid = lax.axis_index("s") * NC + lax.axis_index("c")
        iota = lax.iota(jnp.int32, L)
        slots = [_c16(si) for si in range(12)]
        outs = (o0, o1, o2)
        pltpu.sync_copy(cam_hbm, cam_v)

        @pl.loop(0, UPT)
        def _unit(s):
            u = wid * UPT + s
            n = u // (RB * K)
            rem = u - n * (RB * K)
            r = rem // K
            k = rem - r * K
            cam = [plsc.load_gather(
                       cam_v,
                       [jnp.broadcast_to(n * 3 + c, (L,)).astype(jnp.int32)])
                   for c in range(3)]

            pltpu.sync_copy(pix_hbm.at[n, k, r], pix_v)
            for c in range(3):
                pltpu.sync_copy(bary_hbm.at[n, k, c, r], bary_v.at[c])
            descs = []
            for ci in range(WC):
                for ri in range(8):
                    descs.append(pltpu.async_copy(
                        table_hbm.at[pix_v.at[ci, ri]],
                        rows_v.at[pl.ds((ci * 8 + ri) * 128, 128)], sem))
            for d in descs:
                d.wait()

            @pl.loop(0, WC * 8)
            def _g(i):
                ci = i // 8
                ri = i - ci * 8
                outb = ri * W + ci * 128
                for cb in range(8):
                    rows = iota + (i * 128 + cb * 16)
                    c12 = [plsc.load_gather(rows_v, [rows, slots[si]])
                           for si in range(12)]
                    bw = [bary_v[c, ci, ri, pl.ds(cb * 16, L)]
                          for c in range(3)]
                    d = []
                    for m in range(3):
                        pm = (bw[0] * c12[m] + bw[1] * c12[3 + m]
                              + bw[2] * c12[6 + m])
                        d.append(pm - cam[m])
                    num = c12[9] * d[0] + c12[10] * d[1] + c12[11] * d[2]
                    ss = d[0] * d[0] + d[1] * d[1] + d[2] * d[2]
                    out_v[pl.ds(outb + cb * 16, L)] = num * _rsqrt_clamped(ss)

            p0 = n * (H * W) + r * CP
            for kk in range(3):
                @pl.when(k == kk)
                def _store():
                    pltpu.sync_copy(out_v, outs[kk].at[pl.ds(p0, CP)])

    return shade


def kernel(pix_to_face, bary_coords, verts, faces, cam_origin):
    N, H, W, K = pix_to_face.shape
    V = verts.shape[0]
    F = faces.shape[0]
    NPIX = N * H * W

    gran = NW * 128
    F_pad = ((F + gran - 1) // gran) * gran
    fidx = (jnp.pad(faces, ((0, F_pad - F), (0, 1)))
            .reshape(F_pad // 128, 128, 4).transpose(0, 2, 1))
    cam = jnp.zeros((L,), jnp.float32).at[: N * 3].set(cam_origin.reshape(-1))

    pix6 = (pix_to_face.transpose(0, 3, 1, 2)
            .reshape(N, K, H // 8, 8, W // 128, 128)
            .transpose(0, 1, 2, 4, 3, 5))
    bary7 = (bary_coords.transpose(0, 3, 4, 1, 2)
             .reshape(N, K, 3, H // 8, 8, W // 128, 128)
             .transpose(0, 1, 2, 3, 5, 4, 6))

    VB = ((V + gran - 1) // gran) * gran // 128
    vblk = (jnp.pad(verts, ((0, VB * 128 - V), (0, 1)))
            .reshape(VB, 128, 4).transpose(0, 2, 1))
    verts8 = _vert8(VB)(vblk)
    table = _build_table(V, F_pad)(verts8, fidx)
    o0, o1, o2 = _shade(F_pad, N, K, H, W)(table, pix6, bary7, cam)
    return tuple(o.reshape(N, H, W, 1) for o in (o0, o1, o2))


# speedup vs baseline: 235.5641x; 1.2822x over previous
import functools

import jax
import jax.numpy as jnp
from jax import lax
from jax.experimental import pallas as pl
from jax.experimental.pallas import tpu as pltpu
from jax.experimental.pallas import tpu_sc as plsc

_SC_PARAMS = pltpu.CompilerParams(
    use_tc_tiling_on_sc=False, needs_layout_passes=False)

NC = 2
NS = 16
NW = NC * NS
L = 16

_MAGIC = 0x5F3759DF


def _c16(v, dtype=jnp.int32):
    return jnp.full((L,), v, dtype)


def _b16r(x):
    i = plsc.bitcast(x, jnp.int32)
    return (i + _c16(0x8000)) & _c16(-65536)


def _unpack_pair(w):
    lo = plsc.bitcast(lax.shift_left(w, 16), jnp.float32)
    hi = plsc.bitcast(w & _c16(-65536), jnp.float32)
    return lo, hi


def _rsqrt_clamped(ss):
    x = jnp.maximum(ss, _c16(1e-24, jnp.float32))
    i = plsc.bitcast(x, jnp.int32)
    i = _c16(_MAGIC) - lax.shift_right_logical(i, 1)
    y = plsc.bitcast(i, jnp.float32)
    xh = x * _c16(0.5, jnp.float32)
    th = _c16(1.5, jnp.float32)
    y = y * (th - xh * y * y)
    y = y * (th - xh * y * y)
    y = y * (th - xh * y * y)
    return jnp.minimum(y, _c16(1e12, jnp.float32))


def _vert8(VB):
    BT = VB // NW
    mesh = plsc.VectorSubcoreMesh(
        core_axis_name="c", subcore_axis_name="s", num_cores=NC, num_subcores=NS)

    @functools.partial(
        pl.kernel,
        out_type=jax.ShapeDtypeStruct((VB * 128, 8), jnp.float32),
        mesh=mesh,
        compiler_params=_SC_PARAMS,
        scratch_types=[
            pltpu.VMEM((BT, 4, 128), jnp.float32),
            pltpu.VMEM((BT * 128, 8), jnp.float32),
        ],
    )
    def relayout(vblk_hbm, v8_hbm, blk_v, out_v):
        wid = lax.axis_index("s") * NC + lax.axis_index("c")
        iota = lax.iota(jnp.int32, L)
        cols = [_c16(m) for m in range(3)]
        b0 = wid * BT
        pltpu.sync_copy(vblk_hbm.at[pl.ds(b0, BT)], blk_v)

        @pl.loop(0, BT * 8)
        def _g(g):
            blk = g // 8
            cb = g - blk * 8
            rows = iota + g * L
            for j in range(3):
                val = blk_v[blk, j, pl.ds(cb * L, L)]
                plsc.store_scatter(out_v, [rows, cols[j]], val)

        pltpu.sync_copy(out_v, v8_hbm.at[pl.ds(b0 * 128, BT * 128)])

    return relayout


def _build_table(V, F_pad):
    FT = F_pad // NW
    SUB = 896
    NSUB = FT // SUB
    SUBB = SUB // 128
    NIDX = SUB * 3
    mesh = plsc.VectorSubcoreMesh(
        core_axis_name="c", subcore_axis_name="s", num_cores=NC, num_subcores=NS)

    @functools.partial(
        pl.kernel,
        out_type=jax.ShapeDtypeStruct((F_pad, 8), jnp.int32),
        mesh=mesh,
        compiler_params=_SC_PARAMS,
        scratch_types=[
            pltpu.VMEM((SUBB, 4, 128), jnp.int32),
            pltpu.VMEM((NIDX, 8), jnp.float32),
            pltpu.VMEM((SUB, 8), jnp.int32),
            pltpu.SemaphoreType.DMA,
        ],
    )
    def build(verts_hbm, fidx_hbm, table_hbm, fidx_v, gath_v, row_v, sem):
        wid = lax.axis_index("s") * NC + lax.axis_index("c")
        iota = lax.iota(jnp.int32, L)
        cols = [_c16(m) for m in range(3)]
        slots = [_c16(si) for si in range(12)]

        @pl.loop(0, NSUB)
        def _sub(s):
            f0 = wid * FT + s * SUB
            b0 = f0 // 128
            pltpu.sync_copy(fidx_hbm.at[pl.ds(b0, SUBB)], fidx_v)
            descs = [
                pltpu.async_copy(verts_hbm.at[fidx_v.at[fb, j]],
                                 gath_v.at[pl.ds((fb * 3 + j) * 128, 128)],
                                 sem)
                for fb in range(SUBB) for j in range(3)
            ]
            for d in descs:
                d.wait()

            @pl.loop(0, SUB // L)
            def _g(g):
                base = g * L
                fb = g // 8
                cb = g - fb * 8
                r0 = fb * 384 + cb * L
                v = [[plsc.load_gather(gath_v, [iota + (r0 + j * 128), cols[m]])
                      for m in range(3)] for j in range(3)]
                e1 = [v[1][m] - v[0][m] for m in range(3)]
                e2 = [v[2][m] - v[0][m] for m in range(3)]
                nx = e1[1] * e2[2] - e1[2] * e2[1]
                ny = e1[2] * e2[0] - e1[0] * e2[2]
                nz = e1[0] * e2[1] - e1[1] * e2[0]
                r = _rsqrt_clamped(nx * nx + ny * ny + nz * nz)
                vals = (v[0][0], v[0][1], v[0][2],
                        v[1][0], v[1][1], v[1][2],
                        v[2][0], v[2][1], v[2][2],
                        nx * r, ny * r, nz * r)
                rows = iota + base
                for t in range(6):
                    word = (lax.shift_right_logical(_b16r(vals[2 * t]), 16)
                            | _b16r(vals[2 * t + 1]))
                    plsc.store_scatter(row_v, [rows, slots[t]], word)

            pltpu.sync_copy(row_v, table_hbm.at[pl.ds(f0, SUB)])

    return build


def _shade(F_pad, N, K, H, W):
    NPIX = N * H * W
    RB = H // 8
    WC = W // 128
    CP = 8 * W
    UNITS = N * RB * K
    UPT = UNITS // NW
    mesh = plsc.VectorSubcoreMesh(
        core_axis_name="c", subcore_axis_name="s", num_cores=NC, num_subcores=NS)
    out_sds = jax.ShapeDtypeStruct((NPIX,), jnp.float32)

    @functools.partial(
        pl.kernel,
        out_type=(out_sds, out_sds, out_sds),
        mesh=mesh,
        compiler_params=_SC_PARAMS,
        scratch_types=[
            pltpu.VMEM((2, WC, 8, 128), jnp.int32),
            pltpu.VMEM((CP, 8), jnp.int32),
            pltpu.VMEM((2, 3, WC, 8, 128), jnp.float32),
            pltpu.VMEM((CP,), jnp.float32),
            pltpu.VMEM((L,), jnp.float32),
            pltpu.SemaphoreType.DMA,
        ],
    )
    def shade(table_hbm, pix_hbm, bary_hbm, cam_hbm,
              o0, o1, o2, pix_v, rows_v, bary_v, out_v, cam_v, sem):
        wid = lax.axis_index("s") * NC + lax.axis_index("c")
        iota = lax.iota(jnp.int32, L)
        slots = [_c16(si) for si in range(6)]
        outs = (o0, o1, o2)
        pltpu.sync_copy(cam_hbm, cam_v)

        def unit_nrk(s):
            u = wid * UPT + s
            n = u // (RB * K)
            rem = u - n * (RB * K)
            r = rem // K
            return n, r, rem - r * K

        def load_inputs(s, p):
            n, r, k = unit_nrk(s)
            pltpu.sync_copy(pix_hbm.at[n, k, r], pix_v.at[p])
            for c in range(3):
                pltpu.sync_copy(bary_hbm.at[n, k, c, r], bary_v.at[p, c])

        load_inputs(0, 0)

        @pl.loop(0, UPT)
        def _unit(s):
            p = lax.rem(s, 2)
            n, r, k = unit_nrk(s)
            cam = [plsc.load_gather(
                       cam_v,
                       [jnp.broadcast_to(n * 3 + c, (L,)).astype(jnp.int32)])
                   for c in range(3)]

            descs = []
            for ci in range(WC):
                for ri in range(8):
                    descs.append(pltpu.async_copy(
                        table_hbm.at[pix_v.at[p, ci, ri]],
                        rows_v.at[pl.ds((ci * 8 + ri) * 128, 128)], sem))

            @pl.when(s < UPT - 1)
            def _pf():
                load_inputs(s + 1, 1 - p)

            for ci in range(WC):
                for d in descs[ci * 8:(ci + 1) * 8]:
                    d.wait()

                @pl.loop(0, 8)
                def _g(ri):
                    base = (ci * 8 + ri) * 128
                    outb = ri * W + ci * 128
                    for cb in range(8):
                        rows = iota + (base + cb * 16)
                        pr = [_unpack_pair(
                                  plsc.load_gather(rows_v, [rows, slots[t]]))
                              for t in range(6)]
                        c12 = [x for lohi in pr for x in lohi]
                        bw = [bary_v[p, c, ci, ri, pl.ds(cb * 16, L)]
                              for c in range(3)]
                        d = []
                        for m in range(3):
                            pm = (bw[0] * c12[m] + bw[1] * c12[3 + m]
                                  + bw[2] * c12[6 + m])
                            d.append(pm - cam[m])
                        num = (c12[9] * d[0] + c12[10] * d[1]
                               + c12[11] * d[2])
                        ss = d[0] * d[0] + d[1] * d[1] + d[2] * d[2]
                        out_v[pl.ds(outb + cb * 16, L)] = (
                            num * _rsqrt_clamped(ss))

            p0 = n * (H * W) + r * CP
            for kk in range(3):
                @pl.when(k == kk)
                def _store():
                    pltpu.sync_copy(out_v, outs[kk].at[pl.ds(p0, CP)])

    return shade


def kernel(pix_to_face, bary_coords, verts, faces, cam_origin):
    N, H, W, K = pix_to_face.shape
    V = verts.shape[0]
    F = faces.shape[0]
    NPIX = N * H * W

    gran = NW * 128
    F_pad = ((F + gran - 1) // gran) * gran
    fidx = (jnp.pad(faces, ((0, F_pad - F), (0, 1)))
            .reshape(F_pad // 128, 128, 4).transpose(0, 2, 1))
    cam = jnp.zeros((L,), jnp.float32).at[: N * 3].set(cam_origin.reshape(-1))

    pix6 = (pix_to_face.transpose(0, 3, 1, 2)
            .reshape(N, K, H // 8, 8, W // 128, 128)
            .transpose(0, 1, 2, 4, 3, 5))
    bary7 = (bary_coords.transpose(0, 3, 4, 1, 2)
             .reshape(N, K, 3, H // 8, 8, W // 128, 128)
             .transpose(0, 1, 2, 3, 5, 4, 6))

    VB = ((V + gran - 1) // gran) * gran // 128
    vblk = (jnp.pad(verts, ((0, VB * 128 - V), (0, 1)))
            .reshape(VB, 128, 4).transpose(0, 2, 1))
    verts8 = _vert8(VB)(vblk)
    table = _build_table(V, F_pad)(verts8, fidx)
    o0, o1, o2 = _shade(F_pad, N, K, H, W)(table, pix6, bary7, cam)
    return tuple(o.reshape(N, H, W, 1) for o in (o0, o1, o2))


# speedup vs baseline: 243.2594x; 1.0327x over previous
import functools

import jax
import jax.numpy as jnp
from jax import lax
from jax.experimental import pallas as pl
from jax.experimental.pallas import tpu as pltpu
from jax.experimental.pallas import tpu_sc as plsc

_SC_PARAMS = pltpu.CompilerParams(
    use_tc_tiling_on_sc=False, needs_layout_passes=False)

NC = 2
NS = 16
NW = NC * NS
L = 16

_MAGIC = 0x5F3759DF


def _c16(v, dtype=jnp.int32):
    return jnp.full((L,), v, dtype)


def _b16r(x):
    i = plsc.bitcast(x, jnp.int32)
    return (i + _c16(0x8000)) & _c16(-65536)


def _unpack_pair(w):
    lo = plsc.bitcast(lax.shift_left(w, 16), jnp.float32)
    hi = plsc.bitcast(w & _c16(-65536), jnp.float32)
    return lo, hi


def _rsqrt_clamped(ss):
    x = jnp.maximum(ss, _c16(1e-24, jnp.float32))
    i = plsc.bitcast(x, jnp.int32)
    i = _c16(_MAGIC) - lax.shift_right_logical(i, 1)
    y = plsc.bitcast(i, jnp.float32)
    xh = x * _c16(0.5, jnp.float32)
    th = _c16(1.5, jnp.float32)
    y = y * (th - xh * y * y)
    y = y * (th - xh * y * y)
    y = y * (th - xh * y * y)
    return jnp.minimum(y, _c16(1e12, jnp.float32))


def _vert8(VB):
    BT = VB // NW
    mesh = plsc.VectorSubcoreMesh(
        core_axis_name="c", subcore_axis_name="s", num_cores=NC, num_subcores=NS)

    @functools.partial(
        pl.kernel,
        out_type=jax.ShapeDtypeStruct((VB * 128, 8), jnp.float32),
        mesh=mesh,
        compiler_params=_SC_PARAMS,
        scratch_types=[
            pltpu.VMEM((BT, 4, 128), jnp.float32),
            pltpu.VMEM((BT * 128, 8), jnp.float32),
        ],
    )
    def relayout(vblk_hbm, v8_hbm, blk_v, out_v):
        wid = lax.axis_index("s") * NC + lax.axis_index("c")
        iota = lax.iota(jnp.int32, L)
        cols = [_c16(m) for m in range(3)]
        b0 = wid * BT
        pltpu.sync_copy(vblk_hbm.at[pl.ds(b0, BT)], blk_v)

        @pl.loop(0, BT * 8)
        def _g(g):
            blk = g // 8
            cb = g - blk * 8
            rows = iota + g * L
            for j in range(3):
                val = blk_v[blk, j, pl.ds(cb * L, L)]
                plsc.store_scatter(out_v, [rows, cols[j]], val)

        pltpu.sync_copy(out_v, v8_hbm.at[pl.ds(b0 * 128, BT * 128)])

    return relayout


def _build_table(V, F_pad):
    FT = F_pad // NW
    SUB = 896
    NSUB = FT // SUB
    SUBB = SUB // 128
    NIDX = SUB * 3
    mesh = plsc.VectorSubcoreMesh(
        core_axis_name="c", subcore_axis_name="s", num_cores=NC, num_subcores=NS)

    @functools.partial(
        pl.kernel,
        out_type=jax.ShapeDtypeStruct((F_pad, 8), jnp.int32),
        mesh=mesh,
        compiler_params=_SC_PARAMS,
        scratch_types=[
            pltpu.VMEM((2, SUBB, 4, 128), jnp.int32),
            pltpu.VMEM((NIDX, 8), jnp.float32),
            pltpu.VMEM((SUB, 8), jnp.int32),
            pltpu.SemaphoreType.DMA,
        ],
    )
    def build(verts_hbm, fidx_hbm, table_hbm, fidx_v, gath_v, row_v, sem):
        wid = lax.axis_index("s") * NC + lax.axis_index("c")
        iota = lax.iota(jnp.int32, L)
        cols = [_c16(m) for m in range(3)]
        slots = [_c16(si) for si in range(6)]
        b00 = wid * (FT // 128)
        pltpu.sync_copy(fidx_hbm.at[pl.ds(b00, SUBB)], fidx_v.at[0])

        @pl.loop(0, NSUB)
        def _sub(s):
            p = lax.rem(s, 2)
            f0 = wid * FT + s * SUB
            descs = [
                pltpu.async_copy(verts_hbm.at[fidx_v.at[p, fb, j]],
                                 gath_v.at[pl.ds((fb * 3 + j) * 128, 128)],
                                 sem)
                for fb in range(SUBB) for j in range(3)
            ]

            @pl.when(s < NSUB - 1)
            def _pf():
                pltpu.sync_copy(fidx_hbm.at[pl.ds(b00 + (s + 1) * SUBB, SUBB)],
                                fidx_v.at[1 - p])

            for fb in range(SUBB):
                for d in descs[fb * 3:(fb + 1) * 3]:
                    d.wait()

                @pl.loop(0, 8)
                def _g(cb):
                    r0 = fb * 384 + cb * L
                    v = [[plsc.load_gather(gath_v,
                                           [iota + (r0 + j * 128), cols[m]])
                          for m in range(3)] for j in range(3)]
                    e1 = [v[1][m] - v[0][m] for m in range(3)]
                    e2 = [v[2][m] - v[0][m] for m in range(3)]
                    nx = e1[1] * e2[2] - e1[2] * e2[1]
                    ny = e1[2] * e2[0] - e1[0] * e2[2]
                    nz = e1[0] * e2[1] - e1[1] * e2[0]
                    r = _rsqrt_clamped(nx * nx + ny * ny + nz * nz)
                    vals = (v[0][0], v[0][1], v[0][2],
                            v[1][0], v[1][1], v[1][2],
                            v[2][0], v[2][1], v[2][2],
                            nx * r, ny * r, nz * r)
                    rows = iota + (fb * 128 + cb * L)
                    for t in range(6):
                        word = (lax.shift_right_logical(_b16r(vals[2 * t]), 16)
                                | _b16r(vals[2 * t + 1]))
                        plsc.store_scatter(row_v, [rows, slots[t]], word)

            pltpu.sync_copy(row_v, table_hbm.at[pl.ds(f0, SUB)])

    return build


def _shade(F_pad, N, K, H, W):
    NPIX = N * H * W
    RB = H // 8
    WC = W // 128
    CP = 8 * W
    UNITS = N * RB * K
    UPT = UNITS // NW
    mesh = plsc.VectorSubcoreMesh(
        core_axis_name="c", subcore_axis_name="s", num_cores=NC, num_subcores=NS)
    out_sds = jax.ShapeDtypeStruct((NPIX,), jnp.float32)

    @functools.partial(
        pl.kernel,
        out_type=(out_sds, out_sds, out_sds),
        mesh=mesh,
        compiler_params=_SC_PARAMS,
        scratch_types=[
            pltpu.VMEM((2, WC, 8, 128), jnp.int32),
            pltpu.VMEM((CP, 8), jnp.int32),
            pltpu.VMEM((2, 3, WC, 8, 128), jnp.float32),
            pltpu.VMEM((CP,), jnp.float32),
            pltpu.VMEM((L,), jnp.float32),
            pltpu.SemaphoreType.DMA,
        ],
    )
    def shade(table_hbm, pix_hbm, bary_hbm, cam_hbm,
              o0, o1, o2, pix_v, rows_v, bary_v, out_v, cam_v, sem):
        wid = lax.axis_index("s") * NC + lax.axis_index("c")
        iota = lax.iota(jnp.int32, L)
        slots = [_c16(si) for si in range(6)]
        outs = (o0, o1, o2)
        pltpu.sync_copy(cam_hbm, cam_v)

        def unit_nrk(s):
            u = wid * UPT + s
            n = u // (RB * K)
            rem = u - n * (RB * K)
            r = rem // K
            return n, r, rem - r * K

        def load_inputs(s, p):
            n, r, k = unit_nrk(s)
            pltpu.sync_copy(pix_hbm.at[n, k, r], pix_v.at[p])
            for c in range(3):
                pltpu.sync_copy(bary_hbm.at[n, k, c, r], bary_v.at[p, c])

        load_inputs(0, 0)

        @pl.loop(0, UPT)
        def _unit(s):
            p = lax.rem(s, 2)
            n, r, k = unit_nrk(s)
            cam = [plsc.load_gather(
                       cam_v,
                       [jnp.broadcast_to(n * 3 + c, (L,)).astype(jnp.int32)])
                   for c in range(3)]

            descs = []
            for ci in range(WC):
                for ri in range(8):
                    descs.append(pltpu.async_copy(
                        table_hbm.at[pix_v.at[p, ci, ri]],
                        rows_v.at[pl.ds((ci * 8 + ri) * 128, 128)], sem))

            @pl.when(s < UPT - 1)
            def _pf():
                load_inputs(s + 1, 1 - p)

            for ci in range(WC):
                for d in descs[ci * 8:(ci + 1) * 8]:
                    d.wait()

                @pl.loop(0, 8)
                def _g(ri):
                    base = (ci * 8 + ri) * 128
                    outb = ri * W + ci * 128
                    for cb in range(8):
                        rows = iota + (base + cb * 16)
                        pr = [_unpack_pair(
                                  plsc.load_gather(rows_v, [rows, slots[t]]))
                              for t in range(6)]
                        c12 = [x for lohi in pr for x in lohi]
                        bw = [bary_v[p, c, ci, ri, pl.ds(cb * 16, L)]
                              for c in range(3)]
                        d = []
                        for m in range(3):
                            pm = (bw[0] * c12[m] + bw[1] * c12[3 + m]
                                  + bw[2] * c12[6 + m])
                            d.append(pm - cam[m])
                        num = (c12[9] * d[0] + c12[10] * d[1]
                               + c12[11] * d[2])
                        ss = d[0] * d[0] + d[1] * d[1] + d[2] * d[2]
                        out_v[pl.ds(outb + cb * 16, L)] = (
                            num * _rsqrt_clamped(ss))

            p0 = n * (H * W) + r * CP
            for kk in range(3):
                @pl.when(k == kk)
                def _store():
                    pltpu.sync_copy(out_v, outs[kk].at[pl.ds(p0, CP)])

    return shade


def kernel(pix_to_face, bary_coords, verts, faces, cam_origin):
    N, H, W, K = pix_to_face.shape
    V = verts.shape[0]
    F = faces.shape[0]
    NPIX = N * H * W

    gran = NW * 128
    F_pad = ((F + gran - 1) // gran) * gran
    fidx = (jnp.pad(faces, ((0, F_pad - F), (0, 1)))
            .reshape(F_pad // 128, 128, 4).transpose(0, 2, 1))
    cam = jnp.zeros((L,), jnp.float32).at[: N * 3].set(cam_origin.reshape(-1))

    pix6 = (pix_to_face.transpose(0, 3, 1, 2)
            .reshape(N, K, H // 8, 8, W // 128, 128)
            .transpose(0, 1, 2, 4, 3, 5))
    bary7 = (bary_coords.transpose(0, 3, 4, 1, 2)
             .reshape(N, K, 3, H // 8, 8, W // 128, 128)
             .transpose(0, 1, 2, 3, 5, 4, 6))

    VB = ((V + gran - 1) // gran) * gran // 128
    vblk = (jnp.pad(verts, ((0, VB * 128 - V), (0, 1)))
            .reshape(VB, 128, 4).transpose(0, 2, 1))
    verts8 = _vert8(VB)(vblk)
    table = _build_table(V, F_pad)(verts8, fidx)
    o0, o1, o2 = _shade(F_pad, N, K, H, W)(table, pix6, bary7, cam)
    return tuple(o.reshape(N, H, W, 1) for o in (o0, o1, o2))
